# Initial kernel scaffold; baseline (speedup 1.0000x reference)
import jax
import jax.numpy as jnp
from jax.experimental import pallas as pl


def kernel(x, edge_index, edge_attr, batch, atom_emb, bond_emb, We1, be1, Wn1, bn1, We2, be2, Wn2, bn2):
    raise NotImplementedError("write your pallas kernel here")


# speedup vs baseline: 15.9611x; 15.9611x over previous
import functools

import jax
import jax.numpy as jnp
from jax import lax
from jax.experimental import pallas as pl
from jax.experimental.pallas import tpu as pltpu
from jax.experimental.pallas import tpu_sc as plsc

NC = 2
NS = 16
NW = NC * NS
L = 16

K = 128
KSUP = 8
SUP = K * KSUP


def _cdiv(a, b):
    return (a + b - 1) // b


def _sc_stats(dst3, bond3, s3, z16, npad):
    nsup = dst3.shape[0]
    nit = _cdiv(nsup, NW)
    rows_per_sub = npad // NS
    mesh = plsc.VectorSubcoreMesh(core_axis_name="c", subcore_axis_name="s")

    @functools.partial(
        pl.kernel,
        out_type=jax.ShapeDtypeStruct((NC, npad, L), jnp.float32),
        mesh=mesh,
        compiler_params=pltpu.CompilerParams(use_tc_tiling_on_sc=False, needs_layout_passes=False),
        scratch_types=[
            pltpu.VMEM((2, KSUP, K), jnp.int32),
            pltpu.VMEM((2, KSUP, K), jnp.int32),
            pltpu.VMEM((2, KSUP, K), jnp.float32),
            pltpu.VMEM((K, L), jnp.float32),
            pltpu.VMEM_SHARED((npad, L), jnp.float32),
            pltpu.SemaphoreType.DMA,
        ],
    )
    def body(dst_h, bond_h, s_h, z_h, out_h, dstb, bondb, sb, fbuf, acc, semi):
        c = lax.axis_index("c")
        s = lax.axis_index("s")
        wid = s * NC + c

        pltpu.sync_copy(z_h, acc.at[pl.ds(s * rows_per_sub, rows_per_sub), :])

        iota = lax.broadcasted_iota(jnp.int32, (L,), 0)
        base = jnp.where(iota == 5, 1.0, 0.0).astype(jnp.float32)
        for r in range(K):
            fbuf[r, :] = base

        def issue(ci, buf):
            pltpu.async_copy(dst_h.at[ci], dstb.at[buf], semi)
            pltpu.async_copy(bond_h.at[ci], bondb.at[buf], semi)
            pltpu.async_copy(s_h.at[ci], sb.at[buf], semi)

        issue(wid, 0)

        ones = jnp.ones((L,), jnp.float32)
        zeros = jnp.zeros((L,), jnp.float32)

        def it(t, carry):
            ci = wid + t * NW

            @pl.when(ci < nsup)
            def _():
                cur = lax.rem(t, 2)
                nxt = 1 - cur
                pltpu.make_async_copy(dst_h.at[ci], dstb.at[cur], semi).wait()
                pltpu.make_async_copy(bond_h.at[ci], bondb.at[cur], semi).wait()
                pltpu.make_async_copy(s_h.at[ci], sb.at[cur], semi).wait()

                @pl.when(ci + NW < nsup)
                def _():
                    issue(ci + NW, nxt)

                for j in range(KSUP):
                    for g in range(K // L):
                        rb = g * L + iota
                        bond16 = bondb[cur, j, pl.ds(g * L, L)]
                        s16 = sb[cur, j, pl.ds(g * L, L)]
                        for cc in range(4):
                            plsc.store_scatter(
                                fbuf, [rb, jnp.full((L,), cc, jnp.int32)], zeros)
                        plsc.store_scatter(fbuf, [rb, bond16], ones)
                        plsc.store_scatter(
                            fbuf, [rb, jnp.full((L,), 4, jnp.int32)], s16)
                    pltpu.sync_copy(fbuf, acc.at[dstb.at[cur, j]], add=True)

            return carry

        lax.fori_loop(0, nit, it, 0)
        plsc.subcore_barrier()
        sl = pl.ds(s * rows_per_sub, rows_per_sub)
        pltpu.sync_copy(acc.at[sl, :], out_h.at[c, sl, :])

    return body(dst3, bond3, s3, z16)


def _sc_neighbor_sum(src3, dst3, p, z16, npad):
    nsup = src3.shape[0]
    nit = _cdiv(nsup, NW)
    rows_per_sub = npad // NS
    mesh = plsc.VectorSubcoreMesh(core_axis_name="c", subcore_axis_name="s")

    @functools.partial(
        pl.kernel,
        out_type=jax.ShapeDtypeStruct((NC, npad, L), jnp.float32),
        mesh=mesh,
        compiler_params=pltpu.CompilerParams(use_tc_tiling_on_sc=False, needs_layout_passes=False),
        scratch_types=[
            pltpu.VMEM((2, KSUP, K), jnp.int32),
            pltpu.VMEM((2, KSUP, K), jnp.int32),
            pltpu.VMEM((KSUP, K, L), jnp.float32),
            pltpu.VMEM_SHARED((npad, L), jnp.float32),
            pltpu.SemaphoreType.DMA,
            pltpu.SemaphoreType.DMA,
        ],
    )
    def body(src_h, dst_h, p_h, z_h, out_h, srcb, dstb, rows, acc, semi, semg):
        c = lax.axis_index("c")
        s = lax.axis_index("s")
        wid = s * NC + c

        pltpu.sync_copy(z_h, acc.at[pl.ds(s * rows_per_sub, rows_per_sub), :])

        def issue(ci, buf):
            pltpu.async_copy(src_h.at[ci], srcb.at[buf], semi)
            pltpu.async_copy(dst_h.at[ci], dstb.at[buf], semi)

        issue(wid, 0)

        def it(t, carry):
            ci = wid + t * NW

            @pl.when(ci < nsup)
            def _():
                cur = lax.rem(t, 2)
                nxt = 1 - cur
                pltpu.make_async_copy(src_h.at[ci], srcb.at[cur], semi).wait()
                pltpu.make_async_copy(dst_h.at[ci], dstb.at[cur], semi).wait()

                @pl.when(ci + NW < nsup)
                def _():
                    issue(ci + NW, nxt)

                for k in range(KSUP):
                    pltpu.async_copy(p_h.at[srcb.at[cur, k]], rows.at[k], semg)
                for k in range(KSUP):
                    pltpu.make_async_copy(
                        p_h.at[srcb.at[cur, k]], rows.at[k], semg).wait()
                for k in range(KSUP):
                    pltpu.sync_copy(rows.at[k], acc.at[dstb.at[cur, k]], add=True)

            return carry

        lax.fori_loop(0, nit, it, 0)
        plsc.subcore_barrier()
        sl = pl.ds(s * rows_per_sub, rows_per_sub)
        pltpu.sync_copy(acc.at[sl, :], out_h.at[c, sl, :])

    return body(src3, dst3, p, z16)


def _sc_pool(b3, hg, zp, pg):
    nsup = b3.shape[0]
    nit = _cdiv(nsup, NW)
    rows_per_sub = pg // NS
    mesh = plsc.VectorSubcoreMesh(core_axis_name="c", subcore_axis_name="s")

    @functools.partial(
        pl.kernel,
        out_type=jax.ShapeDtypeStruct((NC, pg, L), jnp.float32),
        mesh=mesh,
        compiler_params=pltpu.CompilerParams(use_tc_tiling_on_sc=False, needs_layout_passes=False),
        scratch_types=[
            pltpu.VMEM((2, KSUP, K), jnp.int32),
            pltpu.VMEM((2, SUP, L), jnp.float32),
            pltpu.VMEM_SHARED((pg, L), jnp.float32),
            pltpu.SemaphoreType.DMA,
        ],
    )
    def body(b_h, hg_h, z_h, out_h, bb, rows, acc, semi):
        c = lax.axis_index("c")
        s = lax.axis_index("s")
        wid = s * NC + c

        pltpu.sync_copy(z_h, acc.at[pl.ds(s * rows_per_sub, rows_per_sub), :])

        def issue(ci, buf):
            pltpu.async_copy(b_h.at[ci], bb.at[buf], semi)
            pltpu.async_copy(
                hg_h.at[pl.ds(ci * SUP, SUP), :], rows.at[buf], semi)

        issue(wid, 0)

        def it(t, carry):
            ci = wid + t * NW

            @pl.when(ci < nsup)
            def _():
                cur = lax.rem(t, 2)
                nxt = 1 - cur
                pltpu.make_async_copy(b_h.at[ci], bb.at[cur], semi).wait()
                pltpu.make_async_copy(
                    hg_h.at[pl.ds(0, SUP), :], rows.at[cur], semi).wait()

                @pl.when(ci + NW < nsup)
                def _():
                    issue(ci + NW, nxt)

                for k in range(KSUP):
                    pltpu.sync_copy(
                        rows.at[cur, pl.ds(k * K, K), :],
                        acc.at[bb.at[cur, k]], add=True)

            return carry

        lax.fori_loop(0, nit, it, 0)
        plsc.subcore_barrier()
        sl = pl.ds(s * rows_per_sub, rows_per_sub)
        pltpu.sync_copy(acc.at[sl, :], out_h.at[c, sl, :])

    return body(b3, hg, zp)


_BN = 3584


def _tc_embed(x3, atom_emb, We1, Wn1, npad, n_atom):
    grid = npad // _BN

    def body(x_ref, ae_ref, we1_ref, wn1_ref, p0_ref, h0n_ref):
        xb = x_ref[0, 0, :]
        oh = (xb[:, None] == lax.broadcasted_iota(
            jnp.int32, (_BN, n_atom), 1)).astype(jnp.float32)
        c0 = jnp.dot(oh, ae_ref[...], preferred_element_type=jnp.float32)
        ad = ae_ref.shape[1]
        p0_ref[...] = jnp.dot(c0, we1_ref[0:ad, :],
                              preferred_element_type=jnp.float32)
        h0n_ref[...] = jnp.dot(c0, wn1_ref[0:ad, :],
                               preferred_element_type=jnp.float32)

    out = pl.pallas_call(
        body,
        grid=(grid,),
        in_specs=[
            pl.BlockSpec((1, 1, _BN), lambda i: (i, 0, 0)),
            pl.BlockSpec(atom_emb.shape, lambda i: (0, 0)),
            pl.BlockSpec(We1.shape, lambda i: (0, 0)),
            pl.BlockSpec(Wn1.shape, lambda i: (0, 0)),
        ],
        out_specs=[
            pl.BlockSpec((_BN, L), lambda i: (i, 0)),
            pl.BlockSpec((_BN, L), lambda i: (i, 0)),
        ],
        out_shape=[
            jax.ShapeDtypeStruct((npad, L), jnp.float32),
            jax.ShapeDtypeStruct((npad, L), jnp.float32),
        ],
    )(x3, atom_emb, We1, Wn1)
    return out


def _tc_layer1(h0n, g0a, g0b, sa, sb, bond_emb, We1, be1r, Wn1, bn1r, We2, Wn2,
               npad, ad):
    grid = npad // _BN

    def body(h0n_ref, g0a_ref, g0b_ref, sa_ref, sb_ref, be_ref, we1_ref,
             be1_ref, wn1_ref, bn1_ref, we2_ref, wn2_ref, p1_ref, h1n_ref):
        st = sa_ref[...] + sb_ref[...]
        b1 = jnp.concatenate([
            jnp.dot(be_ref[...], we1_ref[ad:ad + 4, :],
                    preferred_element_type=jnp.float32),
            we1_ref[ad + 4:ad + 5, :],
            be1_ref[...],
        ], axis=0)
        aggr1 = (g0a_ref[...] + g0b_ref[...]
                 + jnp.dot(st[:, 0:6], b1, preferred_element_type=jnp.float32))
        h1 = jnp.tanh(
            h0n_ref[...]
            + jnp.dot(aggr1, wn1_ref[ad:ad + L, :],
                      preferred_element_type=jnp.float32)
            + bn1_ref[...])
        p1_ref[...] = jnp.dot(h1, we2_ref[0:L, :],
                              preferred_element_type=jnp.float32)
        h1n_ref[...] = jnp.dot(h1, wn2_ref[0:L, :],
                               preferred_element_type=jnp.float32)

    nb = pl.BlockSpec((_BN, L), lambda i: (i, 0))
    full = lambda a: pl.BlockSpec(a.shape, lambda i: tuple(0 for _ in a.shape))
    return pl.pallas_call(
        body,
        grid=(grid,),
        in_specs=[nb, nb, nb, nb, nb, full(bond_emb), full(We1), full(be1r),
                  full(Wn1), full(bn1r), full(We2), full(Wn2)],
        out_specs=[nb, nb],
        out_shape=[
            jax.ShapeDtypeStruct((npad, L), jnp.float32),
            jax.ShapeDtypeStruct((npad, L), jnp.float32),
        ],
    )(h0n, g0a, g0b, sa, sb, bond_emb, We1, be1r, Wn1, bn1r, We2, Wn2)


def _tc_layer2(h1n, g1a, g1b, sa, sb, bond_emb, We2, be2r, Wn2, bn2r, npad):
    grid = npad // _BN

    def body(h1n_ref, g1a_ref, g1b_ref, sa_ref, sb_ref, be_ref, we2_ref,
             be2_ref, wn2_ref, bn2_ref, hg_ref):
        st = sa_ref[...] + sb_ref[...]
        b2 = jnp.concatenate([
            jnp.dot(be_ref[...], we2_ref[L:L + 4, :],
                    preferred_element_type=jnp.float32),
            we2_ref[L + 4:L + 5, :],
            be2_ref[...],
        ], axis=0)
        aggr2 = (g1a_ref[...] + g1b_ref[...]
                 + jnp.dot(st[:, 0:6], b2, preferred_element_type=jnp.float32))
        h2 = (h1n_ref[...]
              + jnp.dot(aggr2, wn2_ref[L:2 * L, :],
                        preferred_element_type=jnp.float32)
              + bn2_ref[...])
        gv = jnp.sum(h2, axis=1) * (1.0 / L)
        col = lax.broadcasted_iota(jnp.int32, (_BN, L), 1)
        hg_ref[...] = jnp.where(col == 0, gv[:, None],
                                jnp.where(col == 1, 1.0, 0.0))

    nb = pl.BlockSpec((_BN, L), lambda i: (i, 0))
    full = lambda a: pl.BlockSpec(a.shape, lambda i: tuple(0 for _ in a.shape))
    return pl.pallas_call(
        body,
        grid=(grid,),
        in_specs=[nb, nb, nb, nb, nb, full(bond_emb), full(We2), full(be2r),
                  full(Wn2), full(bn2r)],
        out_specs=nb,
        out_shape=jax.ShapeDtypeStruct((npad, L), jnp.float32),
    )(h1n, g1a, g1b, sa, sb, bond_emb, We2, be2r, Wn2, bn2r)


def _tc_pool_combine(pa, pb, n_graphs):
    def body(pa_ref, pb_ref, out_ref):
        acc = pa_ref[...] + pb_ref[...]
        col = lax.broadcasted_iota(jnp.int32, acc.shape, 1)
        sums = jnp.sum(jnp.where(col == 0, acc, 0.0), axis=1)
        cnts = jnp.sum(jnp.where(col == 1, acc, 0.0), axis=1)
        out_ref[...] = sums / jnp.maximum(cnts, 1.0)

    return pl.pallas_call(
        body,
        out_shape=jax.ShapeDtypeStruct((n_graphs,), jnp.float32),
    )(pa, pb)


def kernel(x, edge_index, edge_attr, batch, atom_emb, bond_emb,
           We1, be1, Wn1, bn1, We2, be2, Wn2, bn2):
    n = x.shape[0]
    e = edge_index.shape[1]
    n_atom = atom_emb.shape[0]
    ad = atom_emb.shape[1]
    ng = 1024

    npad = _cdiv(n, SUP) * SUP
    epad = _cdiv(e, SUP) * SUP
    pg = 1152

    dummy = jnp.full((epad - e,), n, jnp.int32)
    src3 = jnp.concatenate([edge_index[0], jnp.zeros((epad - e,), jnp.int32)]
                           ).reshape(epad // SUP, KSUP, K)
    dst3 = jnp.concatenate([edge_index[1], dummy]).reshape(epad // SUP, KSUP, K)
    bond3 = jnp.concatenate([edge_attr[:, 0].astype(jnp.int32),
                             jnp.zeros((epad - e,), jnp.int32)]
                            ).reshape(epad // SUP, KSUP, K)
    s3 = jnp.concatenate([edge_attr[:, 1], jnp.zeros((epad - e,), jnp.float32)]
                         ).reshape(epad // SUP, KSUP, K)
    z16 = jnp.zeros((npad // NS, L), jnp.float32)
    x_p = jnp.concatenate([x, jnp.zeros((npad - n,), jnp.int32)])
    x3 = x_p.reshape(npad // _BN, 1, _BN)
    be1r, bn1r = be1.reshape(1, L), bn1.reshape(1, L)
    be2r, bn2r = be2.reshape(1, L), bn2.reshape(1, L)
    batch_p = jnp.concatenate(
        [batch, jnp.full((npad - n,), ng, jnp.int32)]
    ).reshape(npad // SUP, KSUP, K)
    zp = jnp.zeros((pg // NS, L), jnp.float32)

    stats = _sc_stats(dst3, bond3, s3, z16, npad)
    sa, sb = stats[0], stats[1]
    p0, h0n = _tc_embed(x3, atom_emb, We1, Wn1, npad, n_atom)
    g0 = _sc_neighbor_sum(src3, dst3, p0, z16, npad)
    p1, h1n = _tc_layer1(h0n, g0[0], g0[1], sa, sb, bond_emb,
                         We1, be1r, Wn1, bn1r, We2, Wn2, npad, ad)
    g1 = _sc_neighbor_sum(src3, dst3, p1, z16, npad)
    hg = _tc_layer2(h1n, g1[0], g1[1], sa, sb, bond_emb,
                    We2, be2r, Wn2, bn2r, npad)
    pool = _sc_pool(batch_p, hg, zp, pg)
    return _tc_pool_combine(pool[0, 0:ng, :], pool[1, 0:ng, :], ng)
